# initial kernel scaffold (unmeasured)
import jax
import jax.numpy as jnp
from jax import lax
from jax.experimental import pallas as pl
from jax.experimental.pallas import tpu as pltpu

N_DEV = 32
N_EXP = 64
CAP = 6
E_LOC = 2
SLOTS = E_LOC * CAP
N_GSLOT = N_EXP * CAP


def kernel(x, router_W, route_idx, expert_W):
    n_tok, d_model = x.shape
    d_out = expert_W.shape[2]
    del router_W

    def body(x_ref, idx_ref, w_ref, out_ref, g_ref, send_sems, recv_sems):
        me = lax.axis_index("i")

        idx = idx_ref[:, :]
        e_io = lax.broadcasted_iota(jnp.int32, (n_tok, N_EXP), 1)
        onehot = (idx == e_io).astype(jnp.float32)
        i_io = lax.broadcasted_iota(jnp.int32, (n_tok, n_tok), 0)
        j_io = lax.broadcasted_iota(jnp.int32, (n_tok, n_tok), 1)
        lower = (j_io < i_io).astype(jnp.float32)
        counts = jnp.dot(lower, onehot, preferred_element_type=jnp.float32)
        rank = jnp.sum(counts * onehot, axis=1, keepdims=True).astype(jnp.int32)
        valid = rank < CAP
        gslot = jnp.where(valid, idx * CAP + rank, N_GSLOT)

        s_io = lax.broadcasted_iota(jnp.int32, (n_tok, SLOTS), 1)
        qt = (gslot == s_io + me * SLOTS).astype(jnp.bfloat16)
        xb = x_ref[:, :].astype(jnp.bfloat16)
        x_sel = lax.dot_general(
            qt, xb, (((0,), (0,)), ((), ())),
            preferred_element_type=jnp.float32,
        ).astype(jnp.bfloat16)
        wb = w_ref[:, :, :].astype(jnp.bfloat16)
        b0 = jnp.dot(x_sel[0:CAP], wb[0], preferred_element_type=jnp.float32)
        b1 = jnp.dot(x_sel[CAP:SLOTS], wb[1], preferred_element_type=jnp.float32)
        block = jnp.concatenate((b0, b1), axis=0).astype(jnp.bfloat16)
        g_ref[pl.ds(me, 1), :, :] = block.reshape(1, SLOTS, d_out)

        sends = []
        for o in range(1, N_DEV):
            peer = lax.rem(me + o, N_DEV)
            rc = pltpu.make_async_remote_copy(
                src_ref=g_ref.at[me],
                dst_ref=g_ref.at[me],
                send_sem=send_sems.at[o - 1],
                recv_sem=recv_sems.at[N_DEV - o - 1],
                device_id=(peer,),
                device_id_type=pl.DeviceIdType.MESH,
            )
            rc.start()
            sends.append(rc)

        s_io2 = lax.broadcasted_iota(jnp.int32, (n_tok, N_GSLOT), 1)
        p_t = (gslot == s_io2).astype(jnp.bfloat16)

        for o in range(1, N_DEV):
            src_dev = lax.rem(me + N_DEV - o, N_DEV)
            rcv = pltpu.make_async_remote_copy(
                src_ref=g_ref.at[src_dev],
                dst_ref=g_ref.at[src_dev],
                send_sem=send_sems.at[o - 1],
                recv_sem=recv_sems.at[o - 1],
                device_id=(src_dev,),
                device_id_type=pl.DeviceIdType.MESH,
            )
            rcv.wait_recv()

        g_all = g_ref[:, :, :].reshape(N_DEV * SLOTS, d_out)
        out_ref[:, :] = jnp.dot(p_t, g_all, preferred_element_type=jnp.float32)

        for rc in sends:
            rc.wait_send()

    return pl.pallas_call(
        body,
        out_shape=jax.ShapeDtypeStruct((n_tok, d_out), jnp.float32),
        in_specs=[
            pl.BlockSpec(memory_space=pltpu.VMEM),
            pl.BlockSpec(memory_space=pltpu.VMEM),
            pl.BlockSpec(memory_space=pltpu.VMEM),
        ],
        out_specs=pl.BlockSpec(memory_space=pltpu.VMEM),
        scratch_shapes=[
            pltpu.VMEM((N_DEV, SLOTS, d_out), jnp.bfloat16),
            pltpu.SemaphoreType.DMA((N_DEV - 1,)),
            pltpu.SemaphoreType.DMA((N_DEV - 1,)),
        ],
        compiler_params=pltpu.CompilerParams(collective_id=0),
    )(x, route_idx, expert_W)


# baseline (device time: 25533 ns/iter reference)
import jax
import jax.numpy as jnp
from jax import lax
from jax.experimental import pallas as pl
from jax.experimental.pallas import tpu as pltpu

N_DEV = 32
N_EXP = 64
CAP = 6
E_LOC = 2
SLOTS = E_LOC * CAP
N_GSLOT = N_EXP * CAP


def kernel(x, router_W, route_idx, expert_W):
    n_tok, d_model = x.shape
    d_out = expert_W.shape[2]
    del router_W

    def body(x_ref, idx_ref, w_ref, out_ref, g_ref, send_sems, recv_sems):
        me = lax.axis_index("i")

        idx = idx_ref[:, :]
        e_io = lax.broadcasted_iota(jnp.int32, (n_tok, N_EXP), 1)
        onehot = (idx == e_io).astype(jnp.float32)
        i_io = lax.broadcasted_iota(jnp.int32, (n_tok, n_tok), 0)
        j_io = lax.broadcasted_iota(jnp.int32, (n_tok, n_tok), 1)
        lower = (j_io < i_io).astype(jnp.float32)
        counts = jnp.dot(lower, onehot, preferred_element_type=jnp.float32)
        rank = jnp.sum(counts * onehot, axis=1, keepdims=True).astype(jnp.int32)
        valid = rank < CAP
        gslot = jnp.where(valid, idx * CAP + rank, N_GSLOT)

        s_io = lax.broadcasted_iota(jnp.int32, (n_tok, SLOTS), 1)
        qt = (gslot == s_io + me * SLOTS).astype(jnp.bfloat16)
        xb = x_ref[:, :].astype(jnp.bfloat16)
        x_sel = lax.dot_general(
            qt, xb, (((0,), (0,)), ((), ())),
            preferred_element_type=jnp.float32,
        ).astype(jnp.bfloat16)
        wb = w_ref[:, :, :].astype(jnp.bfloat16)
        b0 = jnp.dot(x_sel[0:CAP], wb[0], preferred_element_type=jnp.float32)
        b1 = jnp.dot(x_sel[CAP:SLOTS], wb[1], preferred_element_type=jnp.float32)
        block = jnp.concatenate((b0, b1), axis=0).astype(jnp.bfloat16)
        g_ref[pl.ds(me, 1), :, :] = block.reshape(1, SLOTS, d_out)

        sends = []
        for o in range(1, N_DEV):
            peer = lax.rem(me + o, N_DEV)
            rc = pltpu.make_async_remote_copy(
                src_ref=g_ref.at[me],
                dst_ref=g_ref.at[me],
                send_sem=send_sems.at[o - 1],
                recv_sem=recv_sems.at[o - 1],
                device_id=(peer,),
                device_id_type=pl.DeviceIdType.MESH,
            )
            rc.start()
            sends.append(rc)

        s_io2 = lax.broadcasted_iota(jnp.int32, (n_tok, N_GSLOT), 1)
        p_t = (gslot == s_io2).astype(jnp.bfloat16)

        for o in range(1, N_DEV):
            src_dev = lax.rem(me + N_DEV - o, N_DEV)
            rcv = pltpu.make_async_remote_copy(
                src_ref=g_ref.at[src_dev],
                dst_ref=g_ref.at[src_dev],
                send_sem=send_sems.at[o - 1],
                recv_sem=recv_sems.at[o - 1],
                device_id=(src_dev,),
                device_id_type=pl.DeviceIdType.MESH,
            )
            rcv.wait_recv()

        g_all = g_ref[:, :, :].reshape(N_DEV * SLOTS, d_out)
        out_ref[:, :] = jnp.dot(p_t, g_all, preferred_element_type=jnp.float32)

        for rc in sends:
            rc.wait_send()

    return pl.pallas_call(
        body,
        out_shape=jax.ShapeDtypeStruct((n_tok, d_out), jnp.float32),
        in_specs=[
            pl.BlockSpec(memory_space=pltpu.VMEM),
            pl.BlockSpec(memory_space=pltpu.VMEM),
            pl.BlockSpec(memory_space=pltpu.VMEM),
        ],
        out_specs=pl.BlockSpec(memory_space=pltpu.VMEM),
        scratch_shapes=[
            pltpu.VMEM((N_DEV, SLOTS, d_out), jnp.bfloat16),
            pltpu.SemaphoreType.DMA((N_DEV - 1,)),
            pltpu.SemaphoreType.DMA((N_DEV - 1,)),
        ],
    )(x, route_idx, expert_W)


# device time: 19991 ns/iter; 1.2772x vs baseline; 1.2772x over previous
import jax
import jax.numpy as jnp
from jax import lax
from jax.experimental import pallas as pl
from jax.experimental.pallas import tpu as pltpu

N_DEV = 32
N_EXP = 64
CAP = 6
E_LOC = 2
SLOTS = E_LOC * CAP
N_GSLOT = N_EXP * CAP


def kernel(x, router_W, route_idx, expert_W):
    n_tok, d_model = x.shape
    d_out = expert_W.shape[2]
    del router_W

    def body(x_ref, idx_ref, w_ref, out_ref, g_ref, send_sems, recv_sems):
        me = lax.axis_index("i")

        barrier_sem = pltpu.get_barrier_semaphore()
        for o in range(1, N_DEV):
            pl.semaphore_signal(
                barrier_sem, inc=1,
                device_id=(lax.rem(me + o, N_DEV),),
                device_id_type=pl.DeviceIdType.MESH,
            )

        idx = idx_ref[:, :]
        e_io = lax.broadcasted_iota(jnp.int32, (n_tok, N_EXP), 1)
        onehot = (idx == e_io).astype(jnp.float32)
        i_io = lax.broadcasted_iota(jnp.int32, (n_tok, n_tok), 0)
        j_io = lax.broadcasted_iota(jnp.int32, (n_tok, n_tok), 1)
        lower = (j_io < i_io).astype(jnp.float32)
        counts = jnp.dot(lower, onehot, preferred_element_type=jnp.float32)
        rank = jnp.sum(counts * onehot, axis=1, keepdims=True).astype(jnp.int32)
        valid = rank < CAP
        gslot = jnp.where(valid, idx * CAP + rank, N_GSLOT)

        s_io = lax.broadcasted_iota(jnp.int32, (n_tok, SLOTS), 1)
        qt = (gslot == s_io + me * SLOTS).astype(jnp.bfloat16)
        xb = x_ref[:, :].astype(jnp.bfloat16)
        x_sel = lax.dot_general(
            qt, xb, (((0,), (0,)), ((), ())),
            preferred_element_type=jnp.float32,
        ).astype(jnp.bfloat16)
        wb = w_ref[:, :, :].astype(jnp.bfloat16)
        b0 = jnp.dot(x_sel[0:CAP], wb[0], preferred_element_type=jnp.float32)
        b1 = jnp.dot(x_sel[CAP:SLOTS], wb[1], preferred_element_type=jnp.float32)
        block = jnp.concatenate((b0, b1), axis=0).astype(jnp.bfloat16)
        g_ref[pl.ds(me, 1), :, :] = block.reshape(1, SLOTS, d_out)

        pl.semaphore_wait(barrier_sem, N_DEV - 1)
        sends = []
        for o in range(1, N_DEV):
            peer = lax.rem(me + o, N_DEV)
            rc = pltpu.make_async_remote_copy(
                src_ref=g_ref.at[me],
                dst_ref=g_ref.at[me],
                send_sem=send_sems.at[o - 1],
                recv_sem=recv_sems.at[o - 1],
                device_id=(peer,),
                device_id_type=pl.DeviceIdType.MESH,
            )
            rc.start()
            sends.append(rc)

        s_io2 = lax.broadcasted_iota(jnp.int32, (n_tok, N_GSLOT), 1)
        p_t = (gslot == s_io2).astype(jnp.bfloat16)

        for o in range(1, N_DEV):
            src_dev = lax.rem(me + N_DEV - o, N_DEV)
            rcv = pltpu.make_async_remote_copy(
                src_ref=g_ref.at[src_dev],
                dst_ref=g_ref.at[src_dev],
                send_sem=send_sems.at[o - 1],
                recv_sem=recv_sems.at[o - 1],
                device_id=(src_dev,),
                device_id_type=pl.DeviceIdType.MESH,
            )
            rcv.wait_recv()

        g_all = g_ref[:, :, :].reshape(N_DEV * SLOTS, d_out)
        out_ref[:, :] = jnp.dot(
            p_t, g_all, preferred_element_type=jnp.float32
        ).astype(jnp.bfloat16)

        for rc in sends:
            rc.wait_send()

    return pl.pallas_call(
        body,
        out_shape=jax.ShapeDtypeStruct((n_tok, d_out), jnp.bfloat16),
        in_specs=[
            pl.BlockSpec(memory_space=pltpu.VMEM),
            pl.BlockSpec(memory_space=pltpu.VMEM),
            pl.BlockSpec(memory_space=pltpu.VMEM),
        ],
        out_specs=pl.BlockSpec(memory_space=pltpu.VMEM),
        scratch_shapes=[
            pltpu.VMEM((N_DEV, SLOTS, d_out), jnp.bfloat16),
            pltpu.SemaphoreType.DMA((N_DEV - 1,)),
            pltpu.SemaphoreType.DMA((N_DEV - 1,)),
        ],
        compiler_params=pltpu.CompilerParams(collective_id=0),
    )(x, route_idx, expert_W)
